# baseline (speedup 1.0000x reference)
import functools

import jax
import jax.numpy as jnp
from jax import lax
from jax.experimental import pallas as pl
from jax.experimental.pallas import tpu as pltpu
from jax.experimental.pallas import tpu_sc as plsc

RANK = 32
BATCH = 16384
LANES = 16
RPL = 128 // RANK
NC = 2
NS = 16
NW = NC * NS
BPW = BATCH // NW
CH = 128
NCH = BPW // CH
GPC = CH // LANES
NUM_TIMES = 1000
TLINES = NUM_TIMES // RPL

_mesh = plsc.VectorSubcoreMesh(core_axis_name="c", subcore_axis_name="s")


@functools.partial(
    pl.kernel,
    mesh=_mesh,
    compiler_params=pltpu.CompilerParams(
        needs_layout_passes=False, use_tc_tiling_on_sc=True),
    out_type=jax.ShapeDtypeStruct((BATCH,), jnp.float32),
    scratch_types=[
        pltpu.VMEM((NCH, CH), jnp.int32),
        pltpu.VMEM((NCH, CH), jnp.int32),
        pltpu.VMEM((NCH, CH), jnp.int32),
        pltpu.VMEM((NCH, CH), jnp.int32),
        pltpu.VMEM((NCH, CH), jnp.int32),
        pltpu.VMEM((CH, 128), jnp.float32),
        pltpu.VMEM((CH, 128), jnp.float32),
        pltpu.VMEM((CH, 128), jnp.float32),
        pltpu.VMEM((CH, 128), jnp.float32),
        pltpu.VMEM((TLINES, 128), jnp.float32),
        pltpu.VMEM((BPW,), jnp.float32),
        pltpu.SemaphoreType.DMA,
        pltpu.SemaphoreType.DMA,
        pltpu.SemaphoreType.DMA,
    ],
)
def _hungrey_sc(t_idx_hbm, u_idx_hbm, s_idx_hbm, u_tab, s_tab, t_tab,
                out_hbm, t_idx_v, u_idx_v, s_idx_v, lu_v, ls_v,
                u_buf0, u_buf1, s_buf0, s_buf1, t_lines, out_v,
                sem0, sem1, sem_t):
    wid = lax.axis_index("s") * NC + lax.axis_index("c")
    ibase = wid * NCH

    t_copy = pltpu.make_async_copy(t_tab, t_lines, sem_t)
    t_copy.start()

    pltpu.sync_copy(t_idx_hbm.at[pl.ds(ibase, NCH)], t_idx_v)
    pltpu.sync_copy(u_idx_hbm.at[pl.ds(ibase, NCH)], u_idx_v)
    pltpu.sync_copy(s_idx_hbm.at[pl.ds(ibase, NCH)], s_idx_v)

    for j in range(NCH):
        for v in range(GPC):
            sl = pl.ds(v * LANES, LANES)
            lu_v[j, sl] = u_idx_v[j, sl] >> 2
            ls_v[j, sl] = s_idx_v[j, sl] >> 2

    u_bufs = (u_buf0, u_buf1)
    s_bufs = (s_buf0, s_buf1)
    sems = (sem0, sem1)

    def chunk_copies(j):
        sem = sems[j % 2]
        return (pltpu.make_async_copy(u_tab.at[lu_v.at[j]], u_bufs[j % 2], sem),
                pltpu.make_async_copy(s_tab.at[ls_v.at[j]], s_bufs[j % 2], sem))

    cps = chunk_copies(0)
    for c in cps:
        c.start()
    t_copy.wait()

    lane = lax.iota(jnp.int32, LANES)
    eq = [lane == k for k in range(LANES)]

    for j in range(NCH):
        if j + 1 < NCH:
            nxt = chunk_copies(j + 1)
            for c in nxt:
                c.start()
        for c in cps:
            c.wait()
        if j + 1 < NCH:
            cps = nxt
        u_buf = u_bufs[j % 2]
        s_buf = s_bufs[j % 2]

        def group(g, carry):
            gsl = pl.ds(g * LANES, LANES)
            uvec = u_idx_v[j, gsl]
            svec = s_idx_v[j, gsl]
            tvec = t_idx_v[j, gsl]
            acc = jnp.zeros((LANES,), jnp.float32)
            for k in range(LANES):
                r = g * LANES + k
                u = uvec[k]
                s = svec[k]
                t = tvec[k]
                ou = (u & 3) * RANK
                os_ = (s & 3) * RANK
                tl = t >> 2
                ot = (t & 3) * RANK
                p = (u_buf[r, pl.ds(ou, LANES)]
                     * s_buf[r, pl.ds(os_, LANES)]
                     * t_lines[tl, pl.ds(ot, LANES)]
                     + u_buf[r, pl.ds(ou + LANES, LANES)]
                     * s_buf[r, pl.ds(os_ + LANES, LANES)]
                     * t_lines[tl, pl.ds(ot + LANES, LANES)])
                sv = jnp.broadcast_to(jnp.sum(p), (LANES,))
                acc = jnp.where(eq[k], sv, acc)
            y = 1.0 / (1.0 + jnp.exp(-acc))
            out_v[pl.ds(j * CH + g * LANES, LANES)] = y
            return carry

        lax.fori_loop(0, GPC, group, 0)

    pltpu.sync_copy(out_v, out_hbm.at[pl.ds(wid * BPW, BPW)])


def _tr_body(x_ref, o_ref):
    blk = o_ref.shape[0]
    x = x_ref[...]
    o_ref[...] = jnp.transpose(
        x.reshape(RANK, blk, RPL), (1, 2, 0)).reshape(blk, RPL * RANK)


def _to_lines(emb_t, blk):
    rows = emb_t.shape[1]
    n_lines = rows // RPL
    grid = (n_lines + blk - 1) // blk
    return pl.pallas_call(
        _tr_body,
        grid=(grid,),
        in_specs=[pl.BlockSpec((RANK, blk * RPL), lambda i: (0, i))],
        out_specs=pl.BlockSpec((blk, RPL * RANK), lambda i: (i, 0)),
        out_shape=jax.ShapeDtypeStruct((n_lines, RPL * RANK), jnp.float32),
    )(emb_t)


def kernel(timeIdx, userIdx, servIdx, userEmb, servEmb, timeEmb):
    t_idx = timeIdx.astype(jnp.int32).reshape(BATCH // CH, CH)
    u_idx = userIdx.astype(jnp.int32).reshape(BATCH // CH, CH)
    s_idx = servIdx.astype(jnp.int32).reshape(BATCH // CH, CH)
    u_tab = _to_lines(userEmb.T, 1024)
    s_tab = _to_lines(servEmb.T, 1024)
    t_tab = _to_lines(timeEmb.T, 256)
    return _hungrey_sc(t_idx, u_idx, s_idx, u_tab, s_tab, t_tab)


# speedup vs baseline: 6.2409x; 6.2409x over previous
import functools

import jax
import jax.numpy as jnp
from jax import lax
from jax.experimental import pallas as pl
from jax.experimental.pallas import tpu as pltpu
from jax.experimental.pallas import tpu_sc as plsc

RANK = 32
BATCH = 16384
LANES = 16
RPL = 128 // RANK
NC = 2
NS = 16
NW = NC * NS
BPW = BATCH // NW
CH = 128
NCH = BPW // CH
GPC = CH // LANES
NUM_TIMES = 1000
TLINES = NUM_TIMES // RPL

_mesh = plsc.VectorSubcoreMesh(core_axis_name="c", subcore_axis_name="s")


@functools.partial(
    pl.kernel,
    mesh=_mesh,
    compiler_params=pltpu.CompilerParams(
        needs_layout_passes=False, use_tc_tiling_on_sc=True),
    out_type=jax.ShapeDtypeStruct((BATCH,), jnp.float32),
    scratch_types=[
        pltpu.VMEM((NCH, CH), jnp.int32),
        pltpu.VMEM((NCH, CH), jnp.int32),
        pltpu.VMEM((NCH, CH), jnp.int32),
        pltpu.VMEM((CH, 128), jnp.float32),
        pltpu.VMEM((CH, 128), jnp.float32),
        pltpu.VMEM((CH, 128), jnp.float32),
        pltpu.VMEM((CH, 128), jnp.float32),
        pltpu.VMEM((TLINES, 128), jnp.float32),
        pltpu.VMEM((BPW,), jnp.float32),
        pltpu.SemaphoreType.DMA,
        pltpu.SemaphoreType.DMA,
        pltpu.SemaphoreType.DMA,
    ],
)
def _hungrey_sc(t_idx_hbm, u_idx_hbm, s_idx_hbm, u_tab, s_tab, t_tab,
                out_hbm, t_idx_v, u_idx_v, s_idx_v,
                u_buf0, u_buf1, s_buf0, s_buf1, t_lines, out_v,
                sem0, sem1, sem_t):
    wid = lax.axis_index("s") * NC + lax.axis_index("c")
    ibase = wid * NCH

    t_copy = pltpu.make_async_copy(t_tab, t_lines, sem_t)
    t_copy.start()

    pltpu.sync_copy(t_idx_hbm.at[pl.ds(ibase, NCH)], t_idx_v)
    pltpu.sync_copy(u_idx_hbm.at[pl.ds(ibase, NCH)], u_idx_v)
    pltpu.sync_copy(s_idx_hbm.at[pl.ds(ibase, NCH)], s_idx_v)

    u_bufs = (u_buf0, u_buf1)
    s_bufs = (s_buf0, s_buf1)
    sems = (sem0, sem1)

    def chunk_copies(j):
        sem = sems[j % 2]
        return (pltpu.make_async_copy(u_tab.at[u_idx_v.at[j]], u_bufs[j % 2], sem),
                pltpu.make_async_copy(s_tab.at[s_idx_v.at[j]], s_bufs[j % 2], sem))

    cps = chunk_copies(0)
    for c in cps:
        c.start()
    t_copy.wait()

    lane = lax.iota(jnp.int32, LANES)
    eq = [lane == k for k in range(LANES)]
    lo = pl.ds(0, LANES)
    hi = pl.ds(LANES, LANES)

    for j in range(NCH):
        if j + 1 < NCH:
            nxt = chunk_copies(j + 1)
            for c in nxt:
                c.start()
        for c in cps:
            c.wait()
        if j + 1 < NCH:
            cps = nxt
        u_buf = u_bufs[j % 2]
        s_buf = s_bufs[j % 2]

        def group(g, carry):
            gsl = pl.ds(g * LANES, LANES)
            tvec = t_idx_v[j, gsl]
            acc = jnp.zeros((LANES,), jnp.float32)
            for k in range(LANES):
                r = g * LANES + k
                t = tvec[k]
                tl = t >> 2
                ot = (t & 3) * RANK
                p = (u_buf[r, lo] * s_buf[r, lo] * t_lines[tl, pl.ds(ot, LANES)]
                     + u_buf[r, hi] * s_buf[r, hi]
                     * t_lines[tl, pl.ds(ot + LANES, LANES)])
                sv = jnp.broadcast_to(jnp.sum(p), (LANES,))
                acc = jnp.where(eq[k], sv, acc)
            y = 1.0 / (1.0 + jnp.exp(-acc))
            out_v[pl.ds(j * CH + g * LANES, LANES)] = y
            return carry

        lax.fori_loop(0, GPC, group, 0)

    pltpu.sync_copy(out_v, out_hbm.at[pl.ds(wid * BPW, BPW)])


def _pad_t_body(x_ref, eye_ref, o_ref):
    o_ref[...] = lax.dot_general(
        x_ref[...], eye_ref[...], (((0,), (0,)), ((), ())),
        preferred_element_type=jnp.float32)


def _pad_transpose(emb_t, blk):
    rows = emb_t.shape[1]
    grid = (rows + blk - 1) // blk
    eye = jnp.eye(RANK, 128, dtype=jnp.float32)
    return pl.pallas_call(
        _pad_t_body,
        grid=(grid,),
        in_specs=[pl.BlockSpec((RANK, blk), lambda i: (0, i)),
                  pl.BlockSpec((RANK, 128), lambda i: (0, 0))],
        out_specs=pl.BlockSpec((blk, 128), lambda i: (i, 0)),
        out_shape=jax.ShapeDtypeStruct((rows, 128), jnp.float32),
    )(emb_t, eye)


def kernel(timeIdx, userIdx, servIdx, userEmb, servEmb, timeEmb):
    t_idx = timeIdx.astype(jnp.int32).reshape(BATCH // CH, CH)
    u_idx = userIdx.astype(jnp.int32).reshape(BATCH // CH, CH)
    s_idx = servIdx.astype(jnp.int32).reshape(BATCH // CH, CH)
    u_tab = _pad_transpose(userEmb.T, 8192)
    s_tab = _pad_transpose(servEmb.T, 8192)
    t_tab = timeEmb.reshape(-1, 128)
    return _hungrey_sc(t_idx, u_idx, s_idx, u_tab, s_tab, t_tab)


# speedup vs baseline: 6.6821x; 1.0707x over previous
import functools

import jax
import jax.numpy as jnp
from jax import lax
from jax.experimental import pallas as pl
from jax.experimental.pallas import tpu as pltpu
from jax.experimental.pallas import tpu_sc as plsc

RANK = 32
BATCH = 16384
LANES = 16
RPL = 128 // RANK
NC = 2
NS = 16
NW = NC * NS
BPW = BATCH // NW
CH = 128
NCH = BPW // CH
GPC = CH // LANES
NUM_TIMES = 1000
TLINES = NUM_TIMES // RPL

_mesh = plsc.VectorSubcoreMesh(core_axis_name="c", subcore_axis_name="s")


@functools.partial(
    pl.kernel,
    mesh=_mesh,
    compiler_params=pltpu.CompilerParams(
        needs_layout_passes=False, use_tc_tiling_on_sc=True),
    out_type=jax.ShapeDtypeStruct((BATCH,), jnp.float32),
    scratch_types=[
        pltpu.VMEM((NCH, CH), jnp.int32),
        pltpu.VMEM((NCH, CH), jnp.int32),
        pltpu.VMEM((NCH, CH), jnp.int32),
        pltpu.VMEM((CH, 128), jnp.float32),
        pltpu.VMEM((CH, 128), jnp.float32),
        pltpu.VMEM((CH, 128), jnp.float32),
        pltpu.VMEM((CH, 128), jnp.float32),
        pltpu.VMEM((TLINES, 128), jnp.float32),
        pltpu.VMEM((BPW,), jnp.float32),
        pltpu.SemaphoreType.DMA,
        pltpu.SemaphoreType.DMA,
        pltpu.SemaphoreType.DMA,
    ],
)
def _hungrey_sc(t_idx_hbm, u_idx_hbm, s_idx_hbm, u_tab, s_tab, t_tab,
                out_hbm, t_idx_v, u_idx_v, s_idx_v,
                u_buf0, u_buf1, s_buf0, s_buf1, t_lines, out_v,
                sem0, sem1, sem_t):
    wid = lax.axis_index("s") * NC + lax.axis_index("c")
    ibase = wid * NCH

    t_copy = pltpu.make_async_copy(t_tab, t_lines, sem_t)
    t_copy.start()

    pltpu.sync_copy(t_idx_hbm.at[pl.ds(ibase, NCH)], t_idx_v)
    pltpu.sync_copy(u_idx_hbm.at[pl.ds(ibase, NCH)], u_idx_v)
    pltpu.sync_copy(s_idx_hbm.at[pl.ds(ibase, NCH)], s_idx_v)

    u_bufs = (u_buf0, u_buf1)
    s_bufs = (s_buf0, s_buf1)
    sems = (sem0, sem1)

    def chunk_copies(j):
        sem = sems[j % 2]
        return (pltpu.make_async_copy(u_tab.at[u_idx_v.at[j]], u_bufs[j % 2], sem),
                pltpu.make_async_copy(s_tab.at[s_idx_v.at[j]], s_bufs[j % 2], sem))

    cps = chunk_copies(0)
    for c in cps:
        c.start()
    t_copy.wait()

    lane = lax.iota(jnp.int32, LANES)
    eq = [lane == k for k in range(LANES)]
    lo = pl.ds(0, LANES)
    hi = pl.ds(LANES, LANES)

    for j in range(NCH):
        if j + 1 < NCH:
            nxt = chunk_copies(j + 1)
            for c in nxt:
                c.start()
        for c in cps:
            c.wait()
        if j + 1 < NCH:
            cps = nxt
        u_buf = u_bufs[j % 2]
        s_buf = s_bufs[j % 2]

        def group(g, carry):
            gsl = pl.ds(g * LANES, LANES)
            tvec = t_idx_v[j, gsl]
            acc = jnp.zeros((LANES,), jnp.float32)
            for k in range(LANES):
                r = g * LANES + k
                t = tvec[k]
                tl = t >> 2
                ot = (t & 3) * RANK
                p = (u_buf[r, lo] * s_buf[r, lo] * t_lines[tl, pl.ds(ot, LANES)]
                     + u_buf[r, hi] * s_buf[r, hi]
                     * t_lines[tl, pl.ds(ot + LANES, LANES)])
                sv = jnp.broadcast_to(jnp.sum(p), (LANES,))
                acc = jnp.where(eq[k], sv, acc)
            y = 1.0 / (1.0 + jnp.exp(-acc))
            out_v[pl.ds(j * CH + g * LANES, LANES)] = y
            return carry

        lax.fori_loop(0, GPC, group, 0)

    pltpu.sync_copy(out_v, out_hbm.at[pl.ds(wid * BPW, BPW)])


def _pad_t_body(x_ref, eye_ref, o_ref):
    o_ref[...] = lax.dot_general(
        x_ref[...], eye_ref[...], (((0,), (0,)), ((), ())),
        preferred_element_type=jnp.float32)


def _pad_transpose(emb_t, blk):
    rows = emb_t.shape[1]
    grid = (rows + blk - 1) // blk
    eye = jnp.eye(RANK, 128, dtype=jnp.float32)
    return pl.pallas_call(
        _pad_t_body,
        grid=(grid,),
        in_specs=[pl.BlockSpec((RANK, blk), lambda i: (0, i)),
                  pl.BlockSpec((RANK, 128), lambda i: (0, 0))],
        out_specs=pl.BlockSpec((blk, 128), lambda i: (i, 0)),
        out_shape=jax.ShapeDtypeStruct((rows, 128), jnp.float32),
    )(emb_t, eye)


def kernel(timeIdx, userIdx, servIdx, userEmb, servEmb, timeEmb):
    t_idx = timeIdx.astype(jnp.int32).reshape(BATCH // CH, CH)
    u_idx = userIdx.astype(jnp.int32).reshape(BATCH // CH, CH)
    s_idx = servIdx.astype(jnp.int32).reshape(BATCH // CH, CH)
    u_tab = _pad_transpose(userEmb.T, 16384)
    s_tab = _pad_transpose(servEmb.T, 16384)
    t_tab = timeEmb.reshape(-1, 128)
    return _hungrey_sc(t_idx, u_idx, s_idx, u_tab, s_tab, t_tab)
